# baseline (speedup 1.0000x reference)
import functools

import jax
import jax.numpy as jnp
from jax import lax
from jax.experimental import pallas as pl
from jax.experimental.pallas import tpu as pltpu
from jax.experimental.pallas import tpu_sc as plsc

N = 10000
D = 128
E = 320000

NC = 2
NS = 16
NW = NC * NS
CHUNK = 128
E_PAD = 323584
N_CHUNKS = E_PAD // (NW * CHUNK)
FAST_CID = 1
CF = 96
CS = 62
N_PAD = 10112
ROWS_PER_TILE = N_PAD // NS
DUMMY_ROW = N_PAD - 1
CW = 16

_MESH = plsc.VectorSubcoreMesh(core_axis_name="c", subcore_axis_name="s")
_SC_PARAMS = pltpu.CompilerParams(use_tc_tiling_on_sc=False)


def _zero_fill(acc_sh, zbuf, row0, W):
    zeros16 = jnp.zeros((16,), jnp.float32)
    for i in range(16):
        for j in range(W // 16):
            zbuf[i, pl.ds(j * 16, 16)] = zeros16
    n_full = ROWS_PER_TILE // 16
    for t in range(n_full):
        pltpu.sync_copy(zbuf, acc_sh.at[pl.ds(row0 + t * 16, 16)])
    rem = ROWS_PER_TILE - n_full * 16
    if rem:
        pltpu.sync_copy(zbuf.at[pl.ds(0, rem)],
                        acc_sh.at[pl.ds(row0 + n_full * 16, rem)])


@functools.partial(
    pl.kernel,
    out_type=jax.ShapeDtypeStruct((NC, N_PAD, D), jnp.float32),
    mesh=_MESH,
    compiler_params=_SC_PARAMS,
    scratch_types=[
        pltpu.VMEM((CF, CHUNK), jnp.int32),
        pltpu.VMEM((CF, CHUNK), jnp.int32),
        pltpu.VMEM((CHUNK, D), jnp.float32),
        pltpu.VMEM((16, D), jnp.float32),
        pltpu.VMEM_SHARED((N_PAD, D), jnp.float32),
        pltpu.SemaphoreType.DMA,
    ],
)
def _sc_agg(table_hbm, src_f, dst_f, src_s, dst_s, out_hbm,
            src_v, dst_v, rows_v, zbuf, acc_sh, sem):
    cid = lax.axis_index("c")
    sid = lax.axis_index("s")

    row0 = sid * ROWS_PER_TILE
    _zero_fill(acc_sh, zbuf, row0, D)

    def body(j, carry):
        pltpu.async_copy(table_hbm.at[src_v.at[j]], rows_v, sem).wait()
        pltpu.sync_copy(rows_v, acc_sh.at[dst_v.at[j]], add=True)
        return carry

    @pl.when(cid == FAST_CID)
    def _():
        pltpu.sync_copy(src_f.at[sid], src_v)
        pltpu.sync_copy(dst_f.at[sid], dst_v)
        plsc.subcore_barrier()
        lax.fori_loop(0, CF, body, 0)

    @pl.when(cid != FAST_CID)
    def _():
        pltpu.sync_copy(src_s.at[sid], src_v.at[pl.ds(0, CS)])
        pltpu.sync_copy(dst_s.at[sid], dst_v.at[pl.ds(0, CS)])
        plsc.subcore_barrier()
        lax.fori_loop(0, CS, body, 0)

    plsc.subcore_barrier()

    pltpu.sync_copy(acc_sh.at[pl.ds(row0, ROWS_PER_TILE)],
                    out_hbm.at[cid, pl.ds(row0, ROWS_PER_TILE)])


@functools.partial(
    pl.kernel,
    out_type=jax.ShapeDtypeStruct((NC, N_PAD, CW), jnp.float32),
    mesh=_MESH,
    compiler_params=_SC_PARAMS,
    scratch_types=[
        pltpu.VMEM((N_CHUNKS, CHUNK), jnp.int32),
        pltpu.VMEM((CHUNK, CW), jnp.float32),
        pltpu.VMEM((16, CW), jnp.float32),
        pltpu.VMEM_SHARED((N_PAD, CW), jnp.float32),
    ],
)
def _sc_count(dst_hbm, out_hbm, dst_v, ones_v, zbuf, acc_sh):
    cid = lax.axis_index("c")
    sid = lax.axis_index("s")
    wid = sid * NC + cid

    pltpu.sync_copy(dst_hbm.at[wid], dst_v)

    ones16 = jnp.ones((16,), jnp.float32)
    for i in range(CHUNK):
        ones_v[i, pl.ds(0, 16)] = ones16

    row0 = sid * ROWS_PER_TILE
    _zero_fill(acc_sh, zbuf, row0, CW)
    plsc.subcore_barrier()

    def body(j, carry):
        pltpu.sync_copy(ones_v, acc_sh.at[dst_v.at[j]], add=True)
        return carry

    lax.fori_loop(0, N_CHUNKS, body, 0)
    plsc.subcore_barrier()

    pltpu.sync_copy(acc_sh.at[pl.ds(row0, ROWS_PER_TILE)],
                    out_hbm.at[cid, pl.ds(row0, ROWS_PER_TILE)])


_BLK = 400
_GRID = N // _BLK


def _tc1_body(p_ref, c_ref, x_ref, wl_ref, bl_ref, wr_ref, h_ref, cnt_ref):
    p = p_ref[...]
    agg = p[0] + p[1]
    c = c_ref[...]
    cnt = c[0, :, 0:1] + c[1, :, 0:1]
    cntm = jnp.maximum(cnt, 1.0)
    mean = agg / cntm
    out = (lax.dot_general(mean, wl_ref[...], (((1,), (1,)), ((), ())),
                           preferred_element_type=jnp.float32)
           + bl_ref[...]
           + lax.dot_general(x_ref[...], wr_ref[...], (((1,), (1,)), ((), ())),
                             preferred_element_type=jnp.float32))
    nrm = jnp.maximum(jnp.sqrt(jnp.sum(out * out, axis=1, keepdims=True)), 1e-12)
    h_ref[...] = jnp.maximum(out / nrm, 0.0)
    cnt_ref[...] = cntm


def _tc2_body(p_ref, h_ref, cnt_ref, wl_ref, bl_ref, wr_ref, o_ref):
    p = p_ref[...]
    agg = p[0] + p[1]
    mean = agg / cnt_ref[...]
    out = (lax.dot_general(mean, wl_ref[...], (((1,), (1,)), ((), ())),
                           preferred_element_type=jnp.float32)
           + bl_ref[...]
           + lax.dot_general(h_ref[...], wr_ref[...], (((1,), (1,)), ((), ())),
                             preferred_element_type=jnp.float32))
    nrm = jnp.maximum(jnp.sqrt(jnp.sum(out * out, axis=1, keepdims=True)), 1e-12)
    o_ref[...] = jnp.maximum(out / nrm, 0.0)


def _tc_layer1(p, c, x, Wl, bl, Wr):
    return pl.pallas_call(
        _tc1_body,
        grid=(_GRID,),
        in_specs=[
            pl.BlockSpec((2, _BLK, D), lambda i: (0, i, 0)),
            pl.BlockSpec((2, _BLK, CW), lambda i: (0, i, 0)),
            pl.BlockSpec((_BLK, D), lambda i: (i, 0)),
            pl.BlockSpec((D, D), lambda i: (0, 0)),
            pl.BlockSpec((1, D), lambda i: (0, 0)),
            pl.BlockSpec((D, D), lambda i: (0, 0)),
        ],
        out_specs=[
            pl.BlockSpec((_BLK, D), lambda i: (i, 0)),
            pl.BlockSpec((_BLK, 1), lambda i: (i, 0)),
        ],
        out_shape=[
            jax.ShapeDtypeStruct((N, D), jnp.float32),
            jax.ShapeDtypeStruct((N, 1), jnp.float32),
        ],
    )(p, c, x, Wl, bl, Wr)


def _tc_layer2(p, h, cnt, Wl, bl, Wr):
    return pl.pallas_call(
        _tc2_body,
        grid=(_GRID,),
        in_specs=[
            pl.BlockSpec((2, _BLK, D), lambda i: (0, i, 0)),
            pl.BlockSpec((_BLK, D), lambda i: (i, 0)),
            pl.BlockSpec((_BLK, 1), lambda i: (i, 0)),
            pl.BlockSpec((D, D), lambda i: (0, 0)),
            pl.BlockSpec((1, D), lambda i: (0, 0)),
            pl.BlockSpec((D, D), lambda i: (0, 0)),
        ],
        out_specs=pl.BlockSpec((_BLK, D), lambda i: (i, 0)),
        out_shape=jax.ShapeDtypeStruct((N, D), jnp.float32),
    )(p, h, cnt, Wl, bl, Wr)


@jax.jit
def kernel(x, edge_index, Wl1, bl1, Wr1, Wl2, bl2, Wr2):
    src = edge_index[0]
    dst = edge_index[1]
    pad = E_PAD - E
    src_p = jnp.concatenate([src, jnp.zeros((pad,), jnp.int32)])
    dst_p = jnp.concatenate([dst, jnp.full((pad,), DUMMY_ROW, jnp.int32)])
    dst_r = dst_p.reshape(NW, N_CHUNKS, CHUNK)
    ef = 16 * CF * CHUNK
    src_f = src_p[:ef].reshape(16, CF, CHUNK)
    dst_f = dst_p[:ef].reshape(16, CF, CHUNK)
    src_s = src_p[ef:].reshape(16, CS, CHUNK)
    dst_s = dst_p[ef:].reshape(16, CS, CHUNK)

    c = _sc_count(dst_r)
    p1 = _sc_agg(x, src_f, dst_f, src_s, dst_s)
    h, cnt = _tc_layer1(p1, c, x, Wl1, bl1.reshape(1, D), Wr1)
    p2 = _sc_agg(h, src_f, dst_f, src_s, dst_s)
    return _tc_layer2(p2, h, cnt, Wl2, bl2.reshape(1, D), Wr2)


# speedup vs baseline: 1.0929x; 1.0929x over previous
import functools

import jax
import jax.numpy as jnp
from jax import lax
from jax.experimental import pallas as pl
from jax.experimental.pallas import tpu as pltpu
from jax.experimental.pallas import tpu_sc as plsc

N = 10000
D = 128
E = 320000

NC = 2
NS = 16
NW = NC * NS
CHUNK = 128
E_PAD = 323584
N_CHUNKS = E_PAD // (NW * CHUNK)
FAST_CID = 1
CF = 112
CS = 46
N_PAD = 10112
ROWS_PER_TILE = N_PAD // NS
DUMMY_ROW = N_PAD - 1
CW = 16

_MESH = plsc.VectorSubcoreMesh(core_axis_name="c", subcore_axis_name="s")
_SC_PARAMS = pltpu.CompilerParams(use_tc_tiling_on_sc=False)


def _zero_fill(acc_sh, zbuf, row0, W):
    zeros16 = jnp.zeros((16,), jnp.float32)
    for i in range(16):
        for j in range(W // 16):
            zbuf[i, pl.ds(j * 16, 16)] = zeros16
    n_full = ROWS_PER_TILE // 16
    for t in range(n_full):
        pltpu.sync_copy(zbuf, acc_sh.at[pl.ds(row0 + t * 16, 16)])
    rem = ROWS_PER_TILE - n_full * 16
    if rem:
        pltpu.sync_copy(zbuf.at[pl.ds(0, rem)],
                        acc_sh.at[pl.ds(row0 + n_full * 16, rem)])


@functools.partial(
    pl.kernel,
    out_type=jax.ShapeDtypeStruct((NC, N_PAD, D), jnp.float32),
    mesh=_MESH,
    compiler_params=_SC_PARAMS,
    scratch_types=[
        pltpu.VMEM((CF, CHUNK), jnp.int32),
        pltpu.VMEM((CF, CHUNK), jnp.int32),
        pltpu.VMEM((CHUNK, D), jnp.float32),
        pltpu.VMEM((16, D), jnp.float32),
        pltpu.VMEM_SHARED((N_PAD, D), jnp.float32),
        pltpu.SemaphoreType.DMA,
    ],
)
def _sc_agg(table_hbm, src_f, dst_f, src_s, dst_s, out_hbm,
            src_v, dst_v, rows_v, zbuf, acc_sh, sem):
    cid = lax.axis_index("c")
    sid = lax.axis_index("s")

    row0 = sid * ROWS_PER_TILE
    _zero_fill(acc_sh, zbuf, row0, D)

    def body(j, carry):
        pltpu.async_copy(table_hbm.at[src_v.at[j]], rows_v, sem).wait()
        pltpu.sync_copy(rows_v, acc_sh.at[dst_v.at[j]], add=True)
        return carry

    @pl.when(cid == FAST_CID)
    def _():
        pltpu.sync_copy(src_f.at[sid], src_v)
        pltpu.sync_copy(dst_f.at[sid], dst_v)
        plsc.subcore_barrier()
        lax.fori_loop(0, CF, body, 0)

    @pl.when(cid != FAST_CID)
    def _():
        pltpu.sync_copy(src_s.at[sid], src_v.at[pl.ds(0, CS)])
        pltpu.sync_copy(dst_s.at[sid], dst_v.at[pl.ds(0, CS)])
        plsc.subcore_barrier()
        lax.fori_loop(0, CS, body, 0)

    plsc.subcore_barrier()

    pltpu.sync_copy(acc_sh.at[pl.ds(row0, ROWS_PER_TILE)],
                    out_hbm.at[cid, pl.ds(row0, ROWS_PER_TILE)])


@functools.partial(
    pl.kernel,
    out_type=jax.ShapeDtypeStruct((NC, N_PAD, CW), jnp.float32),
    mesh=_MESH,
    compiler_params=_SC_PARAMS,
    scratch_types=[
        pltpu.VMEM((N_CHUNKS, CHUNK), jnp.int32),
        pltpu.VMEM((CHUNK, CW), jnp.float32),
        pltpu.VMEM((16, CW), jnp.float32),
        pltpu.VMEM_SHARED((N_PAD, CW), jnp.float32),
    ],
)
def _sc_count(dst_hbm, out_hbm, dst_v, ones_v, zbuf, acc_sh):
    cid = lax.axis_index("c")
    sid = lax.axis_index("s")
    wid = sid * NC + cid

    pltpu.sync_copy(dst_hbm.at[wid], dst_v)

    ones16 = jnp.ones((16,), jnp.float32)
    for i in range(CHUNK):
        ones_v[i, pl.ds(0, 16)] = ones16

    row0 = sid * ROWS_PER_TILE
    _zero_fill(acc_sh, zbuf, row0, CW)
    plsc.subcore_barrier()

    def body(j, carry):
        pltpu.sync_copy(ones_v, acc_sh.at[dst_v.at[j]], add=True)
        return carry

    lax.fori_loop(0, N_CHUNKS, body, 0)
    plsc.subcore_barrier()

    pltpu.sync_copy(acc_sh.at[pl.ds(row0, ROWS_PER_TILE)],
                    out_hbm.at[cid, pl.ds(row0, ROWS_PER_TILE)])


_BLK = 400
_GRID = N // _BLK


def _tc1_body(p_ref, c_ref, x_ref, wl_ref, bl_ref, wr_ref, h_ref, cnt_ref):
    p = p_ref[...]
    agg = p[0] + p[1]
    c = c_ref[...]
    cnt = c[0, :, 0:1] + c[1, :, 0:1]
    cntm = jnp.maximum(cnt, 1.0)
    mean = agg / cntm
    out = (lax.dot_general(mean, wl_ref[...], (((1,), (1,)), ((), ())),
                           preferred_element_type=jnp.float32)
           + bl_ref[...]
           + lax.dot_general(x_ref[...], wr_ref[...], (((1,), (1,)), ((), ())),
                             preferred_element_type=jnp.float32))
    nrm = jnp.maximum(jnp.sqrt(jnp.sum(out * out, axis=1, keepdims=True)), 1e-12)
    h_ref[...] = jnp.maximum(out / nrm, 0.0)
    cnt_ref[...] = cntm


def _tc2_body(p_ref, h_ref, cnt_ref, wl_ref, bl_ref, wr_ref, o_ref):
    p = p_ref[...]
    agg = p[0] + p[1]
    mean = agg / cnt_ref[...]
    out = (lax.dot_general(mean, wl_ref[...], (((1,), (1,)), ((), ())),
                           preferred_element_type=jnp.float32)
           + bl_ref[...]
           + lax.dot_general(h_ref[...], wr_ref[...], (((1,), (1,)), ((), ())),
                             preferred_element_type=jnp.float32))
    nrm = jnp.maximum(jnp.sqrt(jnp.sum(out * out, axis=1, keepdims=True)), 1e-12)
    o_ref[...] = jnp.maximum(out / nrm, 0.0)


def _tc_layer1(p, c, x, Wl, bl, Wr):
    return pl.pallas_call(
        _tc1_body,
        grid=(_GRID,),
        in_specs=[
            pl.BlockSpec((2, _BLK, D), lambda i: (0, i, 0)),
            pl.BlockSpec((2, _BLK, CW), lambda i: (0, i, 0)),
            pl.BlockSpec((_BLK, D), lambda i: (i, 0)),
            pl.BlockSpec((D, D), lambda i: (0, 0)),
            pl.BlockSpec((1, D), lambda i: (0, 0)),
            pl.BlockSpec((D, D), lambda i: (0, 0)),
        ],
        out_specs=[
            pl.BlockSpec((_BLK, D), lambda i: (i, 0)),
            pl.BlockSpec((_BLK, 1), lambda i: (i, 0)),
        ],
        out_shape=[
            jax.ShapeDtypeStruct((N, D), jnp.float32),
            jax.ShapeDtypeStruct((N, 1), jnp.float32),
        ],
    )(p, c, x, Wl, bl, Wr)


def _tc_layer2(p, h, cnt, Wl, bl, Wr):
    return pl.pallas_call(
        _tc2_body,
        grid=(_GRID,),
        in_specs=[
            pl.BlockSpec((2, _BLK, D), lambda i: (0, i, 0)),
            pl.BlockSpec((_BLK, D), lambda i: (i, 0)),
            pl.BlockSpec((_BLK, 1), lambda i: (i, 0)),
            pl.BlockSpec((D, D), lambda i: (0, 0)),
            pl.BlockSpec((1, D), lambda i: (0, 0)),
            pl.BlockSpec((D, D), lambda i: (0, 0)),
        ],
        out_specs=pl.BlockSpec((_BLK, D), lambda i: (i, 0)),
        out_shape=jax.ShapeDtypeStruct((N, D), jnp.float32),
    )(p, h, cnt, Wl, bl, Wr)


@jax.jit
def kernel(x, edge_index, Wl1, bl1, Wr1, Wl2, bl2, Wr2):
    src = edge_index[0]
    dst = edge_index[1]
    pad = E_PAD - E
    src_p = jnp.concatenate([src, jnp.zeros((pad,), jnp.int32)])
    dst_p = jnp.concatenate([dst, jnp.full((pad,), DUMMY_ROW, jnp.int32)])
    dst_r = dst_p.reshape(NW, N_CHUNKS, CHUNK)
    ef = 16 * CF * CHUNK
    src_f = src_p[:ef].reshape(16, CF, CHUNK)
    dst_f = dst_p[:ef].reshape(16, CF, CHUNK)
    src_s = src_p[ef:].reshape(16, CS, CHUNK)
    dst_s = dst_p[ef:].reshape(16, CS, CHUNK)

    c = _sc_count(dst_r)
    p1 = _sc_agg(x, src_f, dst_f, src_s, dst_s)
    h, cnt = _tc_layer1(p1, c, x, Wl1, bl1.reshape(1, D), Wr1)
    p2 = _sc_agg(h, src_f, dst_f, src_s, dst_s)
    return _tc_layer2(p2, h, cnt, Wl2, bl2.reshape(1, D), Wr2)


# speedup vs baseline: 1.0947x; 1.0017x over previous
import functools

import jax
import jax.numpy as jnp
from jax import lax
from jax.experimental import pallas as pl
from jax.experimental.pallas import tpu as pltpu
from jax.experimental.pallas import tpu_sc as plsc

N = 10000
D = 128
E = 320000

NC = 2
NS = 16
NW = NC * NS
CHUNK = 128
E_PAD = 323584
N_CHUNKS = E_PAD // (NW * CHUNK)
FAST_CID = 1
CF = 120
CS = 38
N_PAD = 10112
ROWS_PER_TILE = N_PAD // NS
DUMMY_ROW = N_PAD - 1
CW = 16

_MESH = plsc.VectorSubcoreMesh(core_axis_name="c", subcore_axis_name="s")
_SC_PARAMS = pltpu.CompilerParams(use_tc_tiling_on_sc=False)


def _zero_fill(acc_sh, zbuf, row0, W):
    zeros16 = jnp.zeros((16,), jnp.float32)
    for i in range(16):
        for j in range(W // 16):
            zbuf[i, pl.ds(j * 16, 16)] = zeros16
    n_full = ROWS_PER_TILE // 16
    for t in range(n_full):
        pltpu.sync_copy(zbuf, acc_sh.at[pl.ds(row0 + t * 16, 16)])
    rem = ROWS_PER_TILE - n_full * 16
    if rem:
        pltpu.sync_copy(zbuf.at[pl.ds(0, rem)],
                        acc_sh.at[pl.ds(row0 + n_full * 16, rem)])


@functools.partial(
    pl.kernel,
    out_type=jax.ShapeDtypeStruct((NC, N_PAD, D), jnp.float32),
    mesh=_MESH,
    compiler_params=_SC_PARAMS,
    scratch_types=[
        pltpu.VMEM((CF, CHUNK), jnp.int32),
        pltpu.VMEM((CF, CHUNK), jnp.int32),
        pltpu.VMEM((CHUNK, D), jnp.float32),
        pltpu.VMEM((16, D), jnp.float32),
        pltpu.VMEM_SHARED((N_PAD, D), jnp.float32),
        pltpu.SemaphoreType.DMA,
    ],
)
def _sc_agg(table_hbm, src_f, dst_f, src_s, dst_s, out_hbm,
            src_v, dst_v, rows_v, zbuf, acc_sh, sem):
    cid = lax.axis_index("c")
    sid = lax.axis_index("s")

    row0 = sid * ROWS_PER_TILE
    _zero_fill(acc_sh, zbuf, row0, D)

    def body(j, carry):
        pltpu.async_copy(table_hbm.at[src_v.at[j]], rows_v, sem).wait()
        pltpu.sync_copy(rows_v, acc_sh.at[dst_v.at[j]], add=True)
        return carry

    @pl.when(cid == FAST_CID)
    def _():
        pltpu.sync_copy(src_f.at[sid], src_v)
        pltpu.sync_copy(dst_f.at[sid], dst_v)
        plsc.subcore_barrier()
        lax.fori_loop(0, CF, body, 0)

    @pl.when(cid != FAST_CID)
    def _():
        pltpu.sync_copy(src_s.at[sid], src_v.at[pl.ds(0, CS)])
        pltpu.sync_copy(dst_s.at[sid], dst_v.at[pl.ds(0, CS)])
        plsc.subcore_barrier()
        lax.fori_loop(0, CS, body, 0)

    plsc.subcore_barrier()

    pltpu.sync_copy(acc_sh.at[pl.ds(row0, ROWS_PER_TILE)],
                    out_hbm.at[cid, pl.ds(row0, ROWS_PER_TILE)])


@functools.partial(
    pl.kernel,
    out_type=jax.ShapeDtypeStruct((NC, N_PAD, CW), jnp.float32),
    mesh=_MESH,
    compiler_params=_SC_PARAMS,
    scratch_types=[
        pltpu.VMEM((N_CHUNKS, CHUNK), jnp.int32),
        pltpu.VMEM((CHUNK, CW), jnp.float32),
        pltpu.VMEM((16, CW), jnp.float32),
        pltpu.VMEM_SHARED((N_PAD, CW), jnp.float32),
    ],
)
def _sc_count(dst_hbm, out_hbm, dst_v, ones_v, zbuf, acc_sh):
    cid = lax.axis_index("c")
    sid = lax.axis_index("s")
    wid = sid * NC + cid

    pltpu.sync_copy(dst_hbm.at[wid], dst_v)

    ones16 = jnp.ones((16,), jnp.float32)
    for i in range(CHUNK):
        ones_v[i, pl.ds(0, 16)] = ones16

    row0 = sid * ROWS_PER_TILE
    _zero_fill(acc_sh, zbuf, row0, CW)
    plsc.subcore_barrier()

    def body(j, carry):
        pltpu.sync_copy(ones_v, acc_sh.at[dst_v.at[j]], add=True)
        return carry

    lax.fori_loop(0, N_CHUNKS, body, 0)
    plsc.subcore_barrier()

    pltpu.sync_copy(acc_sh.at[pl.ds(row0, ROWS_PER_TILE)],
                    out_hbm.at[cid, pl.ds(row0, ROWS_PER_TILE)])


_BLK = 400
_GRID = N // _BLK


def _tc1_body(p_ref, c_ref, x_ref, wl_ref, bl_ref, wr_ref, h_ref, cnt_ref):
    p = p_ref[...]
    agg = p[0] + p[1]
    c = c_ref[...]
    cnt = c[0, :, 0:1] + c[1, :, 0:1]
    cntm = jnp.maximum(cnt, 1.0)
    mean = agg / cntm
    out = (lax.dot_general(mean, wl_ref[...], (((1,), (1,)), ((), ())),
                           preferred_element_type=jnp.float32)
           + bl_ref[...]
           + lax.dot_general(x_ref[...], wr_ref[...], (((1,), (1,)), ((), ())),
                             preferred_element_type=jnp.float32))
    nrm = jnp.maximum(jnp.sqrt(jnp.sum(out * out, axis=1, keepdims=True)), 1e-12)
    h_ref[...] = jnp.maximum(out / nrm, 0.0)
    cnt_ref[...] = cntm


def _tc2_body(p_ref, h_ref, cnt_ref, wl_ref, bl_ref, wr_ref, o_ref):
    p = p_ref[...]
    agg = p[0] + p[1]
    mean = agg / cnt_ref[...]
    out = (lax.dot_general(mean, wl_ref[...], (((1,), (1,)), ((), ())),
                           preferred_element_type=jnp.float32)
           + bl_ref[...]
           + lax.dot_general(h_ref[...], wr_ref[...], (((1,), (1,)), ((), ())),
                             preferred_element_type=jnp.float32))
    nrm = jnp.maximum(jnp.sqrt(jnp.sum(out * out, axis=1, keepdims=True)), 1e-12)
    o_ref[...] = jnp.maximum(out / nrm, 0.0)


def _tc_layer1(p, c, x, Wl, bl, Wr):
    return pl.pallas_call(
        _tc1_body,
        grid=(_GRID,),
        in_specs=[
            pl.BlockSpec((2, _BLK, D), lambda i: (0, i, 0)),
            pl.BlockSpec((2, _BLK, CW), lambda i: (0, i, 0)),
            pl.BlockSpec((_BLK, D), lambda i: (i, 0)),
            pl.BlockSpec((D, D), lambda i: (0, 0)),
            pl.BlockSpec((1, D), lambda i: (0, 0)),
            pl.BlockSpec((D, D), lambda i: (0, 0)),
        ],
        out_specs=[
            pl.BlockSpec((_BLK, D), lambda i: (i, 0)),
            pl.BlockSpec((_BLK, 1), lambda i: (i, 0)),
        ],
        out_shape=[
            jax.ShapeDtypeStruct((N, D), jnp.float32),
            jax.ShapeDtypeStruct((N, 1), jnp.float32),
        ],
    )(p, c, x, Wl, bl, Wr)


def _tc_layer2(p, h, cnt, Wl, bl, Wr):
    return pl.pallas_call(
        _tc2_body,
        grid=(_GRID,),
        in_specs=[
            pl.BlockSpec((2, _BLK, D), lambda i: (0, i, 0)),
            pl.BlockSpec((_BLK, D), lambda i: (i, 0)),
            pl.BlockSpec((_BLK, 1), lambda i: (i, 0)),
            pl.BlockSpec((D, D), lambda i: (0, 0)),
            pl.BlockSpec((1, D), lambda i: (0, 0)),
            pl.BlockSpec((D, D), lambda i: (0, 0)),
        ],
        out_specs=pl.BlockSpec((_BLK, D), lambda i: (i, 0)),
        out_shape=jax.ShapeDtypeStruct((N, D), jnp.float32),
    )(p, h, cnt, Wl, bl, Wr)


@jax.jit
def kernel(x, edge_index, Wl1, bl1, Wr1, Wl2, bl2, Wr2):
    src = edge_index[0]
    dst = edge_index[1]
    pad = E_PAD - E
    src_p = jnp.concatenate([src, jnp.zeros((pad,), jnp.int32)])
    dst_p = jnp.concatenate([dst, jnp.full((pad,), DUMMY_ROW, jnp.int32)])
    dst_r = dst_p.reshape(NW, N_CHUNKS, CHUNK)
    ef = 16 * CF * CHUNK
    src_f = src_p[:ef].reshape(16, CF, CHUNK)
    dst_f = dst_p[:ef].reshape(16, CF, CHUNK)
    src_s = src_p[ef:].reshape(16, CS, CHUNK)
    dst_s = dst_p[ef:].reshape(16, CS, CHUNK)

    c = _sc_count(dst_r)
    p1 = _sc_agg(x, src_f, dst_f, src_s, dst_s)
    h, cnt = _tc_layer1(p1, c, x, Wl1, bl1.reshape(1, D), Wr1)
    p2 = _sc_agg(h, src_f, dst_f, src_s, dst_s)
    return _tc_layer2(p2, h, cnt, Wl2, bl2.reshape(1, D), Wr2)
